# baseline (speedup 1.0000x reference)
import functools

import jax
import jax.numpy as jnp
from jax import lax
from jax.experimental import pallas as pl
from jax.experimental.pallas import tpu as pltpu
from jax.experimental.pallas import tpu_sc as plsc

_N = 10000
_NPAD = 10240
_RPT = _NPAD // 16
_LASTR = _N - 15 * _RPT
_E = 320000
_TILES = 16
_CHUNK = 512
_CPT = 40
_EPAD = _CHUNK * _CPT * _TILES
_ROWB = 10000
_GRID = _N // _ROWB


def _sc_mesh():
    return plsc.VectorSubcoreMesh(core_axis_name="c", subcore_axis_name="s")


_SC_PARAMS = pltpu.CompilerParams(use_tc_tiling_on_sc=False)


def _make_scatter(d):
    half = jax.ShapeDtypeStruct((_N, d), jnp.float32)

    @functools.partial(
        pl.kernel,
        out_type=(half, half),
        mesh=_sc_mesh(),
        scratch_types=[
            pltpu.MemorySpace.VMEM_SHARED((_NPAD, d), jnp.float32),
            pltpu.VMEM((2, 8, 128), jnp.int32),
            pltpu.VMEM((2, _CHUNK, d), jnp.float32),
            pltpu.SemaphoreType.DMA,
            pltpu.SemaphoreType.DMA,
            pltpu.SemaphoreType.DMA,
            pltpu.SemaphoreType.DMA,
            pltpu.SemaphoreType.DMA,
            pltpu.SemaphoreType.DMA,
        ],
        compiler_params=_SC_PARAMS,
    )
    def scat(hs0, hs1, edges3, acc0, acc1,
             acc_sh, idxbuf, rowbuf, g0, g1, s0, s1, i0, i1):
        t = lax.axis_index("s")
        cid = lax.axis_index("c")
        r0 = t * _RPT

        @pl.when(jnp.logical_and(cid == 0, t < 15))
        def _():
            pltpu.sync_copy(hs0.at[pl.ds(r0, _RPT)], acc_sh.at[pl.ds(r0, _RPT)])

        @pl.when(jnp.logical_and(cid == 1, t < 15))
        def _():
            pltpu.sync_copy(hs1.at[pl.ds(r0, _RPT)], acc_sh.at[pl.ds(r0, _RPT)])

        @pl.when(jnp.logical_and(cid == 0, t == 15))
        def _():
            pltpu.sync_copy(hs0.at[pl.ds(15 * _RPT, _LASTR)],
                            acc_sh.at[pl.ds(15 * _RPT, _LASTR)])

        @pl.when(jnp.logical_and(cid == 1, t == 15))
        def _():
            pltpu.sync_copy(hs1.at[pl.ds(15 * _RPT, _LASTR)],
                            acc_sh.at[pl.ds(15 * _RPT, _LASTR)])

        plsc.subcore_barrier()

        gsems = (g0, g1)
        ssems = (s0, s1)
        isems = (i0, i1)

        def idx_load(c, b):
            pltpu.async_copy(edges3.at[t, c], idxbuf.at[b], isems[b])

        def idx_wait(c, b):
            pltpu.make_async_copy(
                edges3.at[t, c], idxbuf.at[b], isems[b]).wait()

        def run_edges(hs):
            def gather(c, b):
                for j in range(4):
                    pltpu.async_copy(hs.at[idxbuf.at[b, j]],
                                     rowbuf.at[b, pl.ds(j * 128, 128)],
                                     gsems[b])

            def gather_wait(c, b):
                for j in range(4):
                    pltpu.make_async_copy(
                        hs.at[idxbuf.at[b, j]],
                        rowbuf.at[b, pl.ds(j * 128, 128)], gsems[b]).wait()

            def scatter_add(c, b):
                for j in range(4):
                    pltpu.async_copy(rowbuf.at[b, pl.ds(j * 128, 128)],
                                     acc_sh.at[idxbuf.at[b, 4 + j]], ssems[b],
                                     add=True)
                for j in range(4):
                    pltpu.make_async_copy(
                        rowbuf.at[b, pl.ds(j * 128, 128)],
                        acc_sh.at[idxbuf.at[b, 4 + j]], ssems[b]).wait()

            idx_load(0, 0)
            idx_load(1, 1)
            idx_wait(0, 0)
            gather(0, 0)

            @pl.loop(0, _CPT // 2)
            def _(k):
                for b in range(2):
                    c = 2 * k + b

                    @pl.when(c < _CPT - 1)
                    def _():
                        idx_wait(c + 1, 1 - b)
                        gather(c + 1, 1 - b)

                    gather_wait(c, b)
                    scatter_add(c, b)

                    @pl.when(c < _CPT - 2)
                    def _():
                        idx_load(c + 2, b)

        @pl.when(cid == 0)
        def _():
            run_edges(hs0)

        @pl.when(cid == 1)
        def _():
            run_edges(hs1)

        plsc.subcore_barrier()

        @pl.when(cid == 0)
        def _():
            _stage_out(acc_sh, acc0, t, r0)

        @pl.when(cid == 1)
        def _():
            _stage_out(acc_sh, acc1, t, r0)

    return scat


def _stage_out(acc_sh, acc, t, r0):
    @pl.when(t < 15)
    def _():
        pltpu.sync_copy(acc_sh.at[pl.ds(r0, _RPT)], acc.at[pl.ds(r0, _RPT)])

    @pl.when(t == 15)
    def _():
        pltpu.sync_copy(acc_sh.at[pl.ds(15 * _RPT, _LASTR)],
                        acc.at[pl.ds(15 * _RPT, _LASTR)])


def _make_degree():
    out16 = jax.ShapeDtypeStruct((_N, 16), jnp.float32)

    @functools.partial(
        pl.kernel,
        out_type=(out16, out16),
        mesh=_sc_mesh(),
        scratch_types=[
            pltpu.MemorySpace.VMEM_SHARED((_NPAD, 16), jnp.float32),
            pltpu.VMEM((_CPT // 2, 8, 128), jnp.int32),
            pltpu.VMEM((128, 16), jnp.float32),
            pltpu.SemaphoreType.DMA,
        ],
        compiler_params=_SC_PARAMS,
    )
    def degk(edges3, ones16, deg_a, deg_b, deg_sh, idxbuf, onesbuf, asem):
        t = lax.axis_index("s")
        cid = lax.axis_index("c")
        r0 = t * _RPT
        c0 = cid * (_CPT // 2)

        pltpu.sync_copy(ones16.at[pl.ds(r0, _RPT)], deg_sh.at[pl.ds(r0, _RPT)])
        pltpu.sync_copy(ones16.at[pl.ds(0, 128)], onesbuf)
        pltpu.sync_copy(edges3.at[t, pl.ds(c0, _CPT // 2)], idxbuf)
        plsc.subcore_barrier()

        @pl.loop(0, _CPT // 2)
        def _(k):
            for j in range(4):
                pltpu.async_copy(onesbuf, deg_sh.at[idxbuf.at[k, 4 + j]],
                                 asem, add=True)

        @pl.loop(0, _CPT // 2)
        def _(k):
            for j in range(4):
                pltpu.make_async_copy(
                    onesbuf, deg_sh.at[idxbuf.at[k, 4 + j]], asem).wait()

        plsc.subcore_barrier()

        @pl.when(cid == 0)
        def _():
            _stage_out(deg_sh, deg_a, t, r0)

        @pl.when(cid == 1)
        def _():
            _stage_out(deg_sh, deg_b, t, r0)

    return degk


def _row_spec(d):
    return pl.BlockSpec((_ROWB, d), lambda i: (i, 0))


def _full_spec(r, c):
    return pl.BlockSpec((r, c), lambda i: (0, 0))


def _t_first(x, w, deg_a, deg_b):
    def body(x_ref, w_ref, da_ref, db_ref, dinv_ref, hs0_ref, hs1_ref):
        deg = da_ref[...][:, 0:1] + db_ref[...][:, 0:1] - 1.0
        dcol = 1.0 / jnp.sqrt(deg)
        dinv_ref[...] = jnp.broadcast_to(dcol, (_ROWB, 16))
        h = jnp.dot(x_ref[...], w_ref[...], preferred_element_type=jnp.float32)
        hs = h * jnp.broadcast_to(dcol, (_ROWB, 128))
        hs0_ref[...] = hs[:, :64]
        hs1_ref[...] = hs[:, 64:]

    return pl.pallas_call(
        body,
        grid=(_GRID,),
        in_specs=[_row_spec(128), _full_spec(128, 128), _row_spec(16),
                  _row_spec(16)],
        out_specs=[_row_spec(16), _row_spec(64), _row_spec(64)],
        out_shape=[jax.ShapeDtypeStruct((_N, 16), jnp.float32),
                   jax.ShapeDtypeStruct((_N, 64), jnp.float32),
                   jax.ShapeDtypeStruct((_N, 64), jnp.float32)],
    )(x, w, deg_a, deg_b)


def _t_mid(a0, a1, dinv, bvec, res, w, d_next, emit_h):
    d_cur = 2 * a0.shape[1]
    hn = d_next // 2
    has_res = res is not None

    def body(*refs):
        if has_res:
            a0r, a1r, dvr, br, rr, wr = refs[:6]
            outs = refs[6:]
        else:
            a0r, a1r, dvr, br, wr = refs[:5]
            outs = refs[5:]
        u = jnp.concatenate([a0r[...], a1r[...]], axis=1)
        dv = jnp.broadcast_to(dvr[...][:, 0:1], (_ROWB, 128))
        h = jnp.maximum(dv[:, :d_cur] * u + br[...][0:1, :], 0.0)
        if has_res:
            h = h + rr[...]
        if emit_h:
            outs[0][...] = h
            o1, o2 = outs[1], outs[2]
        else:
            o1, o2 = outs[0], outs[1]
        hsn = jnp.dot(h, wr[...], preferred_element_type=jnp.float32)
        hsn = hsn * dv[:, :d_next]
        o1[...] = hsn[:, :hn]
        o2[...] = hsn[:, hn:]

    hd = d_cur // 2
    in_specs = [_row_spec(hd), _row_spec(hd),
                _row_spec(16), _full_spec(8, d_cur)]
    args = [a0, a1, dinv, bvec]
    if has_res:
        in_specs.append(_row_spec(d_cur))
        args.append(res)
    in_specs.append(_full_spec(d_cur, d_next))
    args.append(w)

    out_specs = []
    out_shape = []
    if emit_h:
        out_specs.append(_row_spec(d_cur))
        out_shape.append(jax.ShapeDtypeStruct((_N, d_cur), jnp.float32))
    out_specs += [_row_spec(hn), _row_spec(hn)]
    out_shape += [jax.ShapeDtypeStruct((_N, hn), jnp.float32)] * 2

    return pl.pallas_call(
        body,
        grid=(_GRID,),
        in_specs=in_specs,
        out_specs=out_specs,
        out_shape=out_shape,
    )(*args)


def _t_mha(a0, a1, dinv, be4, wv, bv, wo, bo, wd1):
    def body(a0r, a1r, dvr, ber, wvr, bvr, wor, bor, wdr, z_ref, o1, o2):
        u = jnp.concatenate([a0r[...], a1r[...]], axis=1)
        dv = jnp.broadcast_to(dvr[...][:, 0:1], (_ROWB, 128))
        z = dv[:, :64] * u + ber[...][0:1, :]
        z_ref[...] = z
        za = jnp.dot(z, wvr[...], preferred_element_type=jnp.float32)
        za = za + bvr[...][0:1, :]
        za = jnp.dot(za, wor[...], preferred_element_type=jnp.float32)
        za = za + bor[...][0:1, :]
        hs = jnp.dot(za, wdr[...], preferred_element_type=jnp.float32) * dv
        o1[...] = hs[:, :64]
        o2[...] = hs[:, 64:]

    return pl.pallas_call(
        body,
        grid=(_GRID,),
        in_specs=[_row_spec(32), _row_spec(32),
                  _row_spec(16), _full_spec(8, 64), _full_spec(64, 64),
                  _full_spec(8, 64), _full_spec(64, 64), _full_spec(8, 64),
                  _full_spec(64, 128)],
        out_specs=[_row_spec(64), _row_spec(64), _row_spec(64)],
        out_shape=[jax.ShapeDtypeStruct((_N, 64), jnp.float32),
                   jax.ShapeDtypeStruct((_N, 64), jnp.float32),
                   jax.ShapeDtypeStruct((_N, 64), jnp.float32)],
    )(a0, a1, dinv, be4, wv, bv, wo, bo, wd1)


def _t_last(a0, a1, dinv, bvec):
    def body(a0r, a1r, dvr, br, out_ref):
        u = jnp.concatenate([a0r[...], a1r[...]], axis=1)
        dv = jnp.broadcast_to(dvr[...][:, 0:1], (_ROWB, 128))
        out_ref[...] = dv * u + br[...][0:1, :]

    return pl.pallas_call(
        body,
        grid=(_GRID,),
        in_specs=[_row_spec(64), _row_spec(64),
                  _row_spec(16), _full_spec(8, 128)],
        out_specs=_row_spec(128),
        out_shape=jax.ShapeDtypeStruct((_N, 128), jnp.float32),
    )(a0, a1, dinv, bvec)


def _bb(b):
    return jnp.broadcast_to(b.reshape(1, -1), (8, b.shape[0]))


def kernel(x, edge_index, W_e1, b_e1, W_e2, b_e2, W_e3, b_e3, W_e4, b_e4,
           W_d1, b_d1, W_d2, b_d2, W_d3, b_d3, W_d4, b_d4,
           W_qkv, b_qkv, W_o, b_o):
    src = edge_index[0]
    dst = edge_index[1]
    npad_e = _EPAD // _TILES - _E // _TILES
    ar = jnp.arange(_TILES * npad_e, dtype=jnp.int32)
    pad_s = ((ar * 131) % _N).reshape(_TILES, npad_e)
    pad_d = (_N + (ar * 7) % (_NPAD - _N)).reshape(_TILES, npad_e)
    s2 = jnp.concatenate([src.reshape(_TILES, -1), pad_s], axis=1)
    d2 = jnp.concatenate([dst.reshape(_TILES, -1), pad_d], axis=1)
    edges3 = jnp.concatenate(
        [s2.reshape(_TILES, _CPT, 4, 128), d2.reshape(_TILES, _CPT, 4, 128)],
        axis=2)

    ones16 = jnp.ones((_NPAD, 16), jnp.float32)

    deg_a, deg_b = _make_degree()(edges3, ones16)
    dinv, hs0, hs1 = _t_first(x, W_e1, deg_a, deg_b)

    scat64 = _make_scatter(64)
    scat32 = _make_scatter(32)

    a0, a1 = scat64(hs0, hs1, edges3)
    h1, hs0, hs1 = _t_mid(a0, a1, dinv, _bb(b_e1), None, W_e2, 128, True)
    a0, a1 = scat64(hs0, hs1, edges3)
    h2, hs0, hs1 = _t_mid(a0, a1, dinv, _bb(b_e2), h1, W_e3, 128, True)
    a0, a1 = scat64(hs0, hs1, edges3)
    hs0, hs1 = _t_mid(a0, a1, dinv, _bb(b_e3), h2, W_e4, 64, False)
    a0, a1 = scat32(hs0, hs1, edges3)
    z, hs0, hs1 = _t_mha(a0, a1, dinv, _bb(b_e4),
                         W_qkv[:, 128:], _bb(b_qkv[128:]), W_o, _bb(b_o),
                         W_d1)
    a0, a1 = scat64(hs0, hs1, edges3)
    g1, hs0, hs1 = _t_mid(a0, a1, dinv, _bb(b_d1), None, W_d2, 128, True)
    a0, a1 = scat64(hs0, hs1, edges3)
    g2, hs0, hs1 = _t_mid(a0, a1, dinv, _bb(b_d2), g1, W_d3, 128, True)
    a0, a1 = scat64(hs0, hs1, edges3)
    hs0, hs1 = _t_mid(a0, a1, dinv, _bb(b_d3), g2, W_d4, 128, False)
    a0, a1 = scat64(hs0, hs1, edges3)
    x_recon = _t_last(a0, a1, dinv, _bb(b_d4))
    return (x_recon, z)


# speedup vs baseline: 1.0147x; 1.0147x over previous
import functools

import jax
import jax.numpy as jnp
from jax import lax
from jax.experimental import pallas as pl
from jax.experimental.pallas import tpu as pltpu
from jax.experimental.pallas import tpu_sc as plsc

_N = 10000
_NPAD = 10240
_RPT = _NPAD // 16
_LASTR = _N - 15 * _RPT
_E = 320000
_TILES = 16
_CHUNK = 512
_CPT = 40
_EPAD = _CHUNK * _CPT * _TILES
_ROWB = 5000
_GRID = _N // _ROWB


def _sc_mesh():
    return plsc.VectorSubcoreMesh(core_axis_name="c", subcore_axis_name="s")


_SC_PARAMS = pltpu.CompilerParams(use_tc_tiling_on_sc=False)


def _make_scatter(d):
    half = jax.ShapeDtypeStruct((_N, d), jnp.float32)

    @functools.partial(
        pl.kernel,
        out_type=(half, half),
        mesh=_sc_mesh(),
        scratch_types=[
            pltpu.MemorySpace.VMEM_SHARED((_NPAD, d), jnp.float32),
            pltpu.VMEM((2, 2, 512), jnp.int32),
            pltpu.VMEM((2, _CHUNK, d), jnp.float32),
            pltpu.SemaphoreType.DMA,
            pltpu.SemaphoreType.DMA,
            pltpu.SemaphoreType.DMA,
            pltpu.SemaphoreType.DMA,
            pltpu.SemaphoreType.DMA,
            pltpu.SemaphoreType.DMA,
        ],
        compiler_params=_SC_PARAMS,
    )
    def scat(hs0, hs1, edges3, acc0, acc1,
             acc_sh, idxbuf, rowbuf, g0, g1, s0, s1, i0, i1):
        t = lax.axis_index("s")
        cid = lax.axis_index("c")
        r0 = t * _RPT

        @pl.when(jnp.logical_and(cid == 0, t < 15))
        def _():
            pltpu.sync_copy(hs0.at[pl.ds(r0, _RPT)], acc_sh.at[pl.ds(r0, _RPT)])

        @pl.when(jnp.logical_and(cid == 1, t < 15))
        def _():
            pltpu.sync_copy(hs1.at[pl.ds(r0, _RPT)], acc_sh.at[pl.ds(r0, _RPT)])

        @pl.when(jnp.logical_and(cid == 0, t == 15))
        def _():
            pltpu.sync_copy(hs0.at[pl.ds(15 * _RPT, _LASTR)],
                            acc_sh.at[pl.ds(15 * _RPT, _LASTR)])

        @pl.when(jnp.logical_and(cid == 1, t == 15))
        def _():
            pltpu.sync_copy(hs1.at[pl.ds(15 * _RPT, _LASTR)],
                            acc_sh.at[pl.ds(15 * _RPT, _LASTR)])

        plsc.subcore_barrier()

        gsems = (g0, g1)
        ssems = (s0, s1)
        isems = (i0, i1)

        def idx_load(c, b):
            pltpu.async_copy(edges3.at[t, c], idxbuf.at[b], isems[b])

        def idx_wait(c, b):
            pltpu.make_async_copy(
                edges3.at[t, c], idxbuf.at[b], isems[b]).wait()

        def run_edges(hs):
            def gather(c, b):
                pltpu.async_copy(hs.at[idxbuf.at[b, 0]], rowbuf.at[b],
                                 gsems[b])

            def gather_wait(c, b):
                pltpu.make_async_copy(
                    hs.at[idxbuf.at[b, 0]], rowbuf.at[b], gsems[b]).wait()

            def scatter_add(c, b):
                pltpu.async_copy(rowbuf.at[b], acc_sh.at[idxbuf.at[b, 1]],
                                 ssems[b], add=True)
                pltpu.make_async_copy(
                    rowbuf.at[b], acc_sh.at[idxbuf.at[b, 1]], ssems[b]).wait()

            idx_load(0, 0)
            idx_load(1, 1)
            idx_wait(0, 0)
            gather(0, 0)

            @pl.loop(0, _CPT // 2)
            def _(k):
                for b in range(2):
                    c = 2 * k + b

                    @pl.when(c < _CPT - 1)
                    def _():
                        idx_wait(c + 1, 1 - b)
                        gather(c + 1, 1 - b)

                    gather_wait(c, b)
                    scatter_add(c, b)

                    @pl.when(c < _CPT - 2)
                    def _():
                        idx_load(c + 2, b)

        @pl.when(cid == 0)
        def _():
            run_edges(hs0)

        @pl.when(cid == 1)
        def _():
            run_edges(hs1)

        plsc.subcore_barrier()

        @pl.when(cid == 0)
        def _():
            _stage_out(acc_sh, acc0, t, r0)

        @pl.when(cid == 1)
        def _():
            _stage_out(acc_sh, acc1, t, r0)

    return scat


def _stage_out(acc_sh, acc, t, r0):
    @pl.when(t < 15)
    def _():
        pltpu.sync_copy(acc_sh.at[pl.ds(r0, _RPT)], acc.at[pl.ds(r0, _RPT)])

    @pl.when(t == 15)
    def _():
        pltpu.sync_copy(acc_sh.at[pl.ds(15 * _RPT, _LASTR)],
                        acc.at[pl.ds(15 * _RPT, _LASTR)])


def _make_degree():
    out16 = jax.ShapeDtypeStruct((_N, 16), jnp.float32)

    @functools.partial(
        pl.kernel,
        out_type=(out16, out16),
        mesh=_sc_mesh(),
        scratch_types=[
            pltpu.MemorySpace.VMEM_SHARED((_NPAD, 16), jnp.float32),
            pltpu.VMEM((_CPT // 2, 2, 512), jnp.int32),
            pltpu.VMEM((512, 16), jnp.float32),
            pltpu.SemaphoreType.DMA,
        ],
        compiler_params=_SC_PARAMS,
    )
    def degk(edges3, ones16, deg_a, deg_b, deg_sh, idxbuf, onesbuf, asem):
        t = lax.axis_index("s")
        cid = lax.axis_index("c")
        r0 = t * _RPT
        c0 = cid * (_CPT // 2)

        pltpu.sync_copy(ones16.at[pl.ds(r0, _RPT)], deg_sh.at[pl.ds(r0, _RPT)])
        pltpu.sync_copy(ones16.at[pl.ds(0, 512)], onesbuf)
        pltpu.sync_copy(edges3.at[t, pl.ds(c0, _CPT // 2)], idxbuf)
        plsc.subcore_barrier()

        @pl.loop(0, _CPT // 2)
        def _(k):
            pltpu.async_copy(onesbuf, deg_sh.at[idxbuf.at[k, 1]], asem,
                             add=True)

        @pl.loop(0, _CPT // 2)
        def _(k):
            pltpu.make_async_copy(
                onesbuf, deg_sh.at[idxbuf.at[k, 1]], asem).wait()

        plsc.subcore_barrier()

        @pl.when(cid == 0)
        def _():
            _stage_out(deg_sh, deg_a, t, r0)

        @pl.when(cid == 1)
        def _():
            _stage_out(deg_sh, deg_b, t, r0)

    return degk


def _row_spec(d):
    return pl.BlockSpec((_ROWB, d), lambda i: (i, 0))


def _full_spec(r, c):
    return pl.BlockSpec((r, c), lambda i: (0, 0))


def _t_first(x, w, deg_a, deg_b):
    def body(x_ref, w_ref, da_ref, db_ref, dinv_ref, hs0_ref, hs1_ref):
        deg = da_ref[...][:, 0:1] + db_ref[...][:, 0:1] - 1.0
        dcol = 1.0 / jnp.sqrt(deg)
        dinv_ref[...] = jnp.broadcast_to(dcol, (_ROWB, 16))
        h = jnp.dot(x_ref[...], w_ref[...], preferred_element_type=jnp.float32)
        hs = h * jnp.broadcast_to(dcol, (_ROWB, 128))
        hs0_ref[...] = hs[:, :64]
        hs1_ref[...] = hs[:, 64:]

    return pl.pallas_call(
        body,
        grid=(_GRID,),
        in_specs=[_row_spec(128), _full_spec(128, 128), _row_spec(16),
                  _row_spec(16)],
        out_specs=[_row_spec(16), _row_spec(64), _row_spec(64)],
        out_shape=[jax.ShapeDtypeStruct((_N, 16), jnp.float32),
                   jax.ShapeDtypeStruct((_N, 64), jnp.float32),
                   jax.ShapeDtypeStruct((_N, 64), jnp.float32)],
    )(x, w, deg_a, deg_b)


def _t_mid(a0, a1, dinv, bvec, res, w, d_next, emit_h):
    d_cur = 2 * a0.shape[1]
    hn = d_next // 2
    has_res = res is not None

    def body(*refs):
        if has_res:
            a0r, a1r, dvr, br, rr, wr = refs[:6]
            outs = refs[6:]
        else:
            a0r, a1r, dvr, br, wr = refs[:5]
            outs = refs[5:]
        u = jnp.concatenate([a0r[...], a1r[...]], axis=1)
        dv = jnp.broadcast_to(dvr[...][:, 0:1], (_ROWB, 128))
        h = jnp.maximum(dv[:, :d_cur] * u + br[...][0:1, :], 0.0)
        if has_res:
            h = h + rr[...]
        if emit_h:
            outs[0][...] = h
            o1, o2 = outs[1], outs[2]
        else:
            o1, o2 = outs[0], outs[1]
        hsn = jnp.dot(h, wr[...], preferred_element_type=jnp.float32)
        hsn = hsn * dv[:, :d_next]
        o1[...] = hsn[:, :hn]
        o2[...] = hsn[:, hn:]

    hd = d_cur // 2
    in_specs = [_row_spec(hd), _row_spec(hd),
                _row_spec(16), _full_spec(8, d_cur)]
    args = [a0, a1, dinv, bvec]
    if has_res:
        in_specs.append(_row_spec(d_cur))
        args.append(res)
    in_specs.append(_full_spec(d_cur, d_next))
    args.append(w)

    out_specs = []
    out_shape = []
    if emit_h:
        out_specs.append(_row_spec(d_cur))
        out_shape.append(jax.ShapeDtypeStruct((_N, d_cur), jnp.float32))
    out_specs += [_row_spec(hn), _row_spec(hn)]
    out_shape += [jax.ShapeDtypeStruct((_N, hn), jnp.float32)] * 2

    return pl.pallas_call(
        body,
        grid=(_GRID,),
        in_specs=in_specs,
        out_specs=out_specs,
        out_shape=out_shape,
    )(*args)


def _t_mha(a0, a1, dinv, be4, wv, bv, wo, bo, wd1):
    def body(a0r, a1r, dvr, ber, wvr, bvr, wor, bor, wdr, z_ref, o1, o2):
        u = jnp.concatenate([a0r[...], a1r[...]], axis=1)
        dv = jnp.broadcast_to(dvr[...][:, 0:1], (_ROWB, 128))
        z = dv[:, :64] * u + ber[...][0:1, :]
        z_ref[...] = z
        za = jnp.dot(z, wvr[...], preferred_element_type=jnp.float32)
        za = za + bvr[...][0:1, :]
        za = jnp.dot(za, wor[...], preferred_element_type=jnp.float32)
        za = za + bor[...][0:1, :]
        hs = jnp.dot(za, wdr[...], preferred_element_type=jnp.float32) * dv
        o1[...] = hs[:, :64]
        o2[...] = hs[:, 64:]

    return pl.pallas_call(
        body,
        grid=(_GRID,),
        in_specs=[_row_spec(32), _row_spec(32),
                  _row_spec(16), _full_spec(8, 64), _full_spec(64, 64),
                  _full_spec(8, 64), _full_spec(64, 64), _full_spec(8, 64),
                  _full_spec(64, 128)],
        out_specs=[_row_spec(64), _row_spec(64), _row_spec(64)],
        out_shape=[jax.ShapeDtypeStruct((_N, 64), jnp.float32),
                   jax.ShapeDtypeStruct((_N, 64), jnp.float32),
                   jax.ShapeDtypeStruct((_N, 64), jnp.float32)],
    )(a0, a1, dinv, be4, wv, bv, wo, bo, wd1)


def _t_last(a0, a1, dinv, bvec):
    def body(a0r, a1r, dvr, br, out_ref):
        u = jnp.concatenate([a0r[...], a1r[...]], axis=1)
        dv = jnp.broadcast_to(dvr[...][:, 0:1], (_ROWB, 128))
        out_ref[...] = dv * u + br[...][0:1, :]

    return pl.pallas_call(
        body,
        grid=(_GRID,),
        in_specs=[_row_spec(64), _row_spec(64),
                  _row_spec(16), _full_spec(8, 128)],
        out_specs=_row_spec(128),
        out_shape=jax.ShapeDtypeStruct((_N, 128), jnp.float32),
    )(a0, a1, dinv, bvec)


def _bb(b):
    return jnp.broadcast_to(b.reshape(1, -1), (8, b.shape[0]))


def kernel(x, edge_index, W_e1, b_e1, W_e2, b_e2, W_e3, b_e3, W_e4, b_e4,
           W_d1, b_d1, W_d2, b_d2, W_d3, b_d3, W_d4, b_d4,
           W_qkv, b_qkv, W_o, b_o):
    src = edge_index[0]
    dst = edge_index[1]
    npad_e = _EPAD // _TILES - _E // _TILES
    ar = jnp.arange(_TILES * npad_e, dtype=jnp.int32)
    pad_s = ((ar * 131) % _N).reshape(_TILES, npad_e)
    pad_d = (_N + (ar * 7) % (_NPAD - _N)).reshape(_TILES, npad_e)
    s2 = jnp.concatenate([src.reshape(_TILES, -1), pad_s], axis=1)
    d2 = jnp.concatenate([dst.reshape(_TILES, -1), pad_d], axis=1)
    edges3 = jnp.stack(
        [s2.reshape(_TILES, _CPT, 512), d2.reshape(_TILES, _CPT, 512)],
        axis=2)

    ones16 = jnp.ones((_NPAD, 16), jnp.float32)

    deg_a, deg_b = _make_degree()(edges3, ones16)
    dinv, hs0, hs1 = _t_first(x, W_e1, deg_a, deg_b)

    scat64 = _make_scatter(64)
    scat32 = _make_scatter(32)

    a0, a1 = scat64(hs0, hs1, edges3)
    h1, hs0, hs1 = _t_mid(a0, a1, dinv, _bb(b_e1), None, W_e2, 128, True)
    a0, a1 = scat64(hs0, hs1, edges3)
    h2, hs0, hs1 = _t_mid(a0, a1, dinv, _bb(b_e2), h1, W_e3, 128, True)
    a0, a1 = scat64(hs0, hs1, edges3)
    hs0, hs1 = _t_mid(a0, a1, dinv, _bb(b_e3), h2, W_e4, 64, False)
    a0, a1 = scat32(hs0, hs1, edges3)
    z, hs0, hs1 = _t_mha(a0, a1, dinv, _bb(b_e4),
                         W_qkv[:, 128:], _bb(b_qkv[128:]), W_o, _bb(b_o),
                         W_d1)
    a0, a1 = scat64(hs0, hs1, edges3)
    g1, hs0, hs1 = _t_mid(a0, a1, dinv, _bb(b_d1), None, W_d2, 128, True)
    a0, a1 = scat64(hs0, hs1, edges3)
    g2, hs0, hs1 = _t_mid(a0, a1, dinv, _bb(b_d2), g1, W_d3, 128, True)
    a0, a1 = scat64(hs0, hs1, edges3)
    hs0, hs1 = _t_mid(a0, a1, dinv, _bb(b_d3), g2, W_d4, 128, False)
    a0, a1 = scat64(hs0, hs1, edges3)
    x_recon = _t_last(a0, a1, dinv, _bb(b_d4))
    return (x_recon, z)


# speedup vs baseline: 1.0166x; 1.0019x over previous
import functools

import jax
import jax.numpy as jnp
from jax import lax
from jax.experimental import pallas as pl
from jax.experimental.pallas import tpu as pltpu
from jax.experimental.pallas import tpu_sc as plsc

_N = 10000
_NPAD = 10240
_RPT = _NPAD // 16
_LASTR = _N - 15 * _RPT
_E = 320000
_TILES = 16
_CHUNK = 512
_CPT = 40
_EPAD = _CHUNK * _CPT * _TILES
_ROWB = 5000
_GRID = _N // _ROWB


def _sc_mesh():
    return plsc.VectorSubcoreMesh(core_axis_name="c", subcore_axis_name="s")


_SC_PARAMS = pltpu.CompilerParams(use_tc_tiling_on_sc=False)


def _make_scatter(d):
    half = jax.ShapeDtypeStruct((_N, d), jnp.float32)

    @functools.partial(
        pl.kernel,
        out_type=(half, half),
        mesh=_sc_mesh(),
        scratch_types=[
            pltpu.MemorySpace.VMEM_SHARED((_NPAD, d), jnp.float32),
            pltpu.VMEM((2, 8, 128), jnp.int32),
            pltpu.VMEM((2, _CHUNK, d), jnp.float32),
            pltpu.SemaphoreType.DMA,
            pltpu.SemaphoreType.DMA,
            pltpu.SemaphoreType.DMA,
            pltpu.SemaphoreType.DMA,
            pltpu.SemaphoreType.DMA,
            pltpu.SemaphoreType.DMA,
        ],
        compiler_params=_SC_PARAMS,
    )
    def scat(hs0, hs1, edges3, acc0, acc1,
             acc_sh, idxbuf, rowbuf, g0, g1, s0, s1, i0, i1):
        t = lax.axis_index("s")
        cid = lax.axis_index("c")
        r0 = t * _RPT

        @pl.when(jnp.logical_and(cid == 0, t < 15))
        def _():
            pltpu.sync_copy(hs0.at[pl.ds(r0, _RPT)], acc_sh.at[pl.ds(r0, _RPT)])

        @pl.when(jnp.logical_and(cid == 1, t < 15))
        def _():
            pltpu.sync_copy(hs1.at[pl.ds(r0, _RPT)], acc_sh.at[pl.ds(r0, _RPT)])

        @pl.when(jnp.logical_and(cid == 0, t == 15))
        def _():
            pltpu.sync_copy(hs0.at[pl.ds(15 * _RPT, _LASTR)],
                            acc_sh.at[pl.ds(15 * _RPT, _LASTR)])

        @pl.when(jnp.logical_and(cid == 1, t == 15))
        def _():
            pltpu.sync_copy(hs1.at[pl.ds(15 * _RPT, _LASTR)],
                            acc_sh.at[pl.ds(15 * _RPT, _LASTR)])

        plsc.subcore_barrier()

        gsems = (g0, g1)
        ssems = (s0, s1)
        isems = (i0, i1)

        def idx_load(c, b):
            pltpu.async_copy(edges3.at[t, c], idxbuf.at[b], isems[b])

        def idx_wait(c, b):
            pltpu.make_async_copy(
                edges3.at[t, c], idxbuf.at[b], isems[b]).wait()

        def run_edges(hs):
            def gather(c, b):
                for j in range(4):
                    pltpu.async_copy(hs.at[idxbuf.at[b, j]],
                                     rowbuf.at[b, pl.ds(j * 128, 128)],
                                     gsems[b])

            def gather_wait(c, b):
                for j in range(4):
                    pltpu.make_async_copy(
                        hs.at[idxbuf.at[b, j]],
                        rowbuf.at[b, pl.ds(j * 128, 128)], gsems[b]).wait()

            def scatter_add(c, b):
                for j in range(4):
                    pltpu.async_copy(rowbuf.at[b, pl.ds(j * 128, 128)],
                                     acc_sh.at[idxbuf.at[b, 4 + j]], ssems[b],
                                     add=True)
                for j in range(4):
                    pltpu.make_async_copy(
                        rowbuf.at[b, pl.ds(j * 128, 128)],
                        acc_sh.at[idxbuf.at[b, 4 + j]], ssems[b]).wait()

            idx_load(0, 0)
            idx_load(1, 1)
            idx_wait(0, 0)
            gather(0, 0)

            @pl.loop(0, _CPT // 2)
            def _(k):
                for b in range(2):
                    c = 2 * k + b

                    @pl.when(c < _CPT - 1)
                    def _():
                        idx_wait(c + 1, 1 - b)
                        gather(c + 1, 1 - b)

                    gather_wait(c, b)
                    scatter_add(c, b)

                    @pl.when(c < _CPT - 2)
                    def _():
                        idx_load(c + 2, b)

        @pl.when(cid == 0)
        def _():
            run_edges(hs0)

        @pl.when(cid == 1)
        def _():
            run_edges(hs1)

        plsc.subcore_barrier()

        @pl.when(cid == 0)
        def _():
            _stage_out(acc_sh, acc0, t, r0)

        @pl.when(cid == 1)
        def _():
            _stage_out(acc_sh, acc1, t, r0)

    return scat


def _stage_out(acc_sh, acc, t, r0):
    @pl.when(t < 15)
    def _():
        pltpu.sync_copy(acc_sh.at[pl.ds(r0, _RPT)], acc.at[pl.ds(r0, _RPT)])

    @pl.when(t == 15)
    def _():
        pltpu.sync_copy(acc_sh.at[pl.ds(15 * _RPT, _LASTR)],
                        acc.at[pl.ds(15 * _RPT, _LASTR)])


def _make_degree():
    out16 = jax.ShapeDtypeStruct((_N, 16), jnp.float32)

    @functools.partial(
        pl.kernel,
        out_type=(out16, out16),
        mesh=_sc_mesh(),
        scratch_types=[
            pltpu.MemorySpace.VMEM_SHARED((_NPAD, 16), jnp.float32),
            pltpu.VMEM((_CPT // 2, 8, 128), jnp.int32),
            pltpu.VMEM((128, 16), jnp.float32),
            pltpu.SemaphoreType.DMA,
        ],
        compiler_params=_SC_PARAMS,
    )
    def degk(edges3, ones16, deg_a, deg_b, deg_sh, idxbuf, onesbuf, asem):
        t = lax.axis_index("s")
        cid = lax.axis_index("c")
        r0 = t * _RPT
        c0 = cid * (_CPT // 2)

        pltpu.sync_copy(ones16.at[pl.ds(r0, _RPT)], deg_sh.at[pl.ds(r0, _RPT)])
        pltpu.sync_copy(ones16.at[pl.ds(0, 128)], onesbuf)
        pltpu.sync_copy(edges3.at[t, pl.ds(c0, _CPT // 2)], idxbuf)
        plsc.subcore_barrier()

        @pl.loop(0, _CPT // 2)
        def _(k):
            for j in range(4):
                pltpu.async_copy(onesbuf, deg_sh.at[idxbuf.at[k, 4 + j]],
                                 asem, add=True)

        @pl.loop(0, _CPT // 2)
        def _(k):
            for j in range(4):
                pltpu.make_async_copy(
                    onesbuf, deg_sh.at[idxbuf.at[k, 4 + j]], asem).wait()

        plsc.subcore_barrier()

        @pl.when(cid == 0)
        def _():
            _stage_out(deg_sh, deg_a, t, r0)

        @pl.when(cid == 1)
        def _():
            _stage_out(deg_sh, deg_b, t, r0)

    return degk


def _row_spec(d):
    return pl.BlockSpec((_ROWB, d), lambda i: (i, 0))


def _full_spec(r, c):
    return pl.BlockSpec((r, c), lambda i: (0, 0))


def _t_matmul(x, w):
    def body(x_ref, w_ref, h_ref):
        h_ref[...] = jnp.dot(x_ref[...], w_ref[...],
                             preferred_element_type=jnp.float32)

    return pl.pallas_call(
        body,
        grid=(_GRID,),
        in_specs=[_row_spec(128), _full_spec(128, 128)],
        out_specs=_row_spec(128),
        out_shape=jax.ShapeDtypeStruct((_N, 128), jnp.float32),
    )(x, w)


def _t_first(h, deg_a, deg_b):
    def body(h_ref, da_ref, db_ref, dinv_ref, hs0_ref, hs1_ref):
        deg = da_ref[...][:, 0:1] + db_ref[...][:, 0:1] - 1.0
        dcol = 1.0 / jnp.sqrt(deg)
        dinv_ref[...] = jnp.broadcast_to(dcol, (_ROWB, 16))
        hs = h_ref[...] * jnp.broadcast_to(dcol, (_ROWB, 128))
        hs0_ref[...] = hs[:, :64]
        hs1_ref[...] = hs[:, 64:]

    return pl.pallas_call(
        body,
        grid=(_GRID,),
        in_specs=[_row_spec(128), _row_spec(16), _row_spec(16)],
        out_specs=[_row_spec(16), _row_spec(64), _row_spec(64)],
        out_shape=[jax.ShapeDtypeStruct((_N, 16), jnp.float32),
                   jax.ShapeDtypeStruct((_N, 64), jnp.float32),
                   jax.ShapeDtypeStruct((_N, 64), jnp.float32)],
    )(h, deg_a, deg_b)


def _t_mid(a0, a1, dinv, bvec, res, w, d_next, emit_h):
    d_cur = 2 * a0.shape[1]
    hn = d_next // 2
    has_res = res is not None

    def body(*refs):
        if has_res:
            a0r, a1r, dvr, br, rr, wr = refs[:6]
            outs = refs[6:]
        else:
            a0r, a1r, dvr, br, wr = refs[:5]
            outs = refs[5:]
        u = jnp.concatenate([a0r[...], a1r[...]], axis=1)
        dv = jnp.broadcast_to(dvr[...][:, 0:1], (_ROWB, 128))
        h = jnp.maximum(dv[:, :d_cur] * u + br[...][0:1, :], 0.0)
        if has_res:
            h = h + rr[...]
        if emit_h:
            outs[0][...] = h
            o1, o2 = outs[1], outs[2]
        else:
            o1, o2 = outs[0], outs[1]
        hsn = jnp.dot(h, wr[...], preferred_element_type=jnp.float32)
        hsn = hsn * dv[:, :d_next]
        o1[...] = hsn[:, :hn]
        o2[...] = hsn[:, hn:]

    hd = d_cur // 2
    in_specs = [_row_spec(hd), _row_spec(hd),
                _row_spec(16), _full_spec(8, d_cur)]
    args = [a0, a1, dinv, bvec]
    if has_res:
        in_specs.append(_row_spec(d_cur))
        args.append(res)
    in_specs.append(_full_spec(d_cur, d_next))
    args.append(w)

    out_specs = []
    out_shape = []
    if emit_h:
        out_specs.append(_row_spec(d_cur))
        out_shape.append(jax.ShapeDtypeStruct((_N, d_cur), jnp.float32))
    out_specs += [_row_spec(hn), _row_spec(hn)]
    out_shape += [jax.ShapeDtypeStruct((_N, hn), jnp.float32)] * 2

    return pl.pallas_call(
        body,
        grid=(_GRID,),
        in_specs=in_specs,
        out_specs=out_specs,
        out_shape=out_shape,
    )(*args)


def _t_mha(a0, a1, dinv, be4, wv, bv, wo, bo, wd1):
    def body(a0r, a1r, dvr, ber, wvr, bvr, wor, bor, wdr, z_ref, o1, o2):
        u = jnp.concatenate([a0r[...], a1r[...]], axis=1)
        dv = jnp.broadcast_to(dvr[...][:, 0:1], (_ROWB, 128))
        z = dv[:, :64] * u + ber[...][0:1, :]
        z_ref[...] = z
        za = jnp.dot(z, wvr[...], preferred_element_type=jnp.float32)
        za = za + bvr[...][0:1, :]
        za = jnp.dot(za, wor[...], preferred_element_type=jnp.float32)
        za = za + bor[...][0:1, :]
        hs = jnp.dot(za, wdr[...], preferred_element_type=jnp.float32) * dv
        o1[...] = hs[:, :64]
        o2[...] = hs[:, 64:]

    return pl.pallas_call(
        body,
        grid=(_GRID,),
        in_specs=[_row_spec(32), _row_spec(32),
                  _row_spec(16), _full_spec(8, 64), _full_spec(64, 64),
                  _full_spec(8, 64), _full_spec(64, 64), _full_spec(8, 64),
                  _full_spec(64, 128)],
        out_specs=[_row_spec(64), _row_spec(64), _row_spec(64)],
        out_shape=[jax.ShapeDtypeStruct((_N, 64), jnp.float32),
                   jax.ShapeDtypeStruct((_N, 64), jnp.float32),
                   jax.ShapeDtypeStruct((_N, 64), jnp.float32)],
    )(a0, a1, dinv, be4, wv, bv, wo, bo, wd1)


def _t_last(a0, a1, dinv, bvec):
    def body(a0r, a1r, dvr, br, out_ref):
        u = jnp.concatenate([a0r[...], a1r[...]], axis=1)
        dv = jnp.broadcast_to(dvr[...][:, 0:1], (_ROWB, 128))
        out_ref[...] = dv * u + br[...][0:1, :]

    return pl.pallas_call(
        body,
        grid=(_GRID,),
        in_specs=[_row_spec(64), _row_spec(64),
                  _row_spec(16), _full_spec(8, 128)],
        out_specs=_row_spec(128),
        out_shape=jax.ShapeDtypeStruct((_N, 128), jnp.float32),
    )(a0, a1, dinv, bvec)


def _bb(b):
    return jnp.broadcast_to(b.reshape(1, -1), (8, b.shape[0]))


def kernel(x, edge_index, W_e1, b_e1, W_e2, b_e2, W_e3, b_e3, W_e4, b_e4,
           W_d1, b_d1, W_d2, b_d2, W_d3, b_d3, W_d4, b_d4,
           W_qkv, b_qkv, W_o, b_o):
    src = edge_index[0]
    dst = edge_index[1]
    npad_e = _EPAD // _TILES - _E // _TILES
    ar = jnp.arange(_TILES * npad_e, dtype=jnp.int32)
    pad_s = ((ar * 131) % _N).reshape(_TILES, npad_e)
    pad_d = (_N + (ar * 7) % (_NPAD - _N)).reshape(_TILES, npad_e)
    s2 = jnp.concatenate([src.reshape(_TILES, -1), pad_s], axis=1)
    d2 = jnp.concatenate([dst.reshape(_TILES, -1), pad_d], axis=1)
    edges3 = jnp.concatenate(
        [s2.reshape(_TILES, _CPT, 4, 128), d2.reshape(_TILES, _CPT, 4, 128)],
        axis=2)

    ones16 = jnp.ones((_NPAD, 16), jnp.float32)

    deg_a, deg_b = _make_degree()(edges3, ones16)
    h0 = _t_matmul(x, W_e1)
    dinv, hs0, hs1 = _t_first(h0, deg_a, deg_b)

    scat64 = _make_scatter(64)
    scat32 = _make_scatter(32)

    a0, a1 = scat64(hs0, hs1, edges3)
    h1, hs0, hs1 = _t_mid(a0, a1, dinv, _bb(b_e1), None, W_e2, 128, True)
    a0, a1 = scat64(hs0, hs1, edges3)
    h2, hs0, hs1 = _t_mid(a0, a1, dinv, _bb(b_e2), h1, W_e3, 128, True)
    a0, a1 = scat64(hs0, hs1, edges3)
    hs0, hs1 = _t_mid(a0, a1, dinv, _bb(b_e3), h2, W_e4, 64, False)
    a0, a1 = scat32(hs0, hs1, edges3)
    z, hs0, hs1 = _t_mha(a0, a1, dinv, _bb(b_e4),
                         W_qkv[:, 128:], _bb(b_qkv[128:]), W_o, _bb(b_o),
                         W_d1)
    a0, a1 = scat64(hs0, hs1, edges3)
    g1, hs0, hs1 = _t_mid(a0, a1, dinv, _bb(b_d1), None, W_d2, 128, True)
    a0, a1 = scat64(hs0, hs1, edges3)
    g2, hs0, hs1 = _t_mid(a0, a1, dinv, _bb(b_d2), g1, W_d3, 128, True)
    a0, a1 = scat64(hs0, hs1, edges3)
    hs0, hs1 = _t_mid(a0, a1, dinv, _bb(b_d3), g2, W_d4, 128, False)
    a0, a1 = scat64(hs0, hs1, edges3)
    x_recon = _t_last(a0, a1, dinv, _bb(b_d4))
    return (x_recon, z)


# speedup vs baseline: 1.0237x; 1.0070x over previous
import functools

import jax
import jax.numpy as jnp
from jax import lax
from jax.experimental import pallas as pl
from jax.experimental.pallas import tpu as pltpu
from jax.experimental.pallas import tpu_sc as plsc

_N = 10000
_NPAD = 10240
_RPT = _NPAD // 16
_LASTR = _N - 15 * _RPT
_E = 320000
_TILES = 16
_CHUNK = 512
_CPT = 40
_EPAD = _CHUNK * _CPT * _TILES
_ROWB = 5000
_GRID = _N // _ROWB


def _sc_mesh():
    return plsc.VectorSubcoreMesh(core_axis_name="c", subcore_axis_name="s")


_SC_PARAMS = pltpu.CompilerParams(use_tc_tiling_on_sc=False)


def _make_scatter(d):
    half = jax.ShapeDtypeStruct((_N, d), jnp.float32)

    @functools.partial(
        pl.kernel,
        out_type=(half, half),
        mesh=_sc_mesh(),
        scratch_types=[
            pltpu.MemorySpace.VMEM_SHARED((_NPAD, d), jnp.float32),
            pltpu.VMEM((2, 8, 128), jnp.int32),
            pltpu.VMEM((2, _CHUNK, d), jnp.float32),
            pltpu.SemaphoreType.DMA,
            pltpu.SemaphoreType.DMA,
            pltpu.SemaphoreType.DMA,
            pltpu.SemaphoreType.DMA,
            pltpu.SemaphoreType.DMA,
            pltpu.SemaphoreType.DMA,
        ],
        compiler_params=_SC_PARAMS,
    )
    def scat(hs0, hs1, edges3, acc0, acc1,
             acc_sh, idxbuf, rowbuf, g0, g1, s0, s1, i0, i1):
        t = lax.axis_index("s")
        cid = lax.axis_index("c")
        r0 = t * _RPT

        pltpu.async_copy(edges3.at[t, 0], idxbuf.at[0], i0)
        pltpu.async_copy(edges3.at[t, 1], idxbuf.at[1], i1)

        @pl.when(jnp.logical_and(cid == 0, t < 15))
        def _():
            pltpu.sync_copy(hs0.at[pl.ds(r0, _RPT)], acc_sh.at[pl.ds(r0, _RPT)])

        @pl.when(jnp.logical_and(cid == 1, t < 15))
        def _():
            pltpu.sync_copy(hs1.at[pl.ds(r0, _RPT)], acc_sh.at[pl.ds(r0, _RPT)])

        @pl.when(jnp.logical_and(cid == 0, t == 15))
        def _():
            pltpu.sync_copy(hs0.at[pl.ds(15 * _RPT, _LASTR)],
                            acc_sh.at[pl.ds(15 * _RPT, _LASTR)])

        @pl.when(jnp.logical_and(cid == 1, t == 15))
        def _():
            pltpu.sync_copy(hs1.at[pl.ds(15 * _RPT, _LASTR)],
                            acc_sh.at[pl.ds(15 * _RPT, _LASTR)])

        gsems = (g0, g1)
        ssems = (s0, s1)
        isems = (i0, i1)

        def idx_load(c, b):
            pltpu.async_copy(edges3.at[t, c], idxbuf.at[b], isems[b])

        def idx_wait(c, b):
            pltpu.make_async_copy(
                edges3.at[t, c], idxbuf.at[b], isems[b]).wait()

        def run_edges(hs):
            def gather(c, b):
                for j in range(4):
                    pltpu.async_copy(hs.at[idxbuf.at[b, j]],
                                     rowbuf.at[b, pl.ds(j * 128, 128)],
                                     gsems[b])

            def gather_wait(c, b):
                for j in range(4):
                    pltpu.make_async_copy(
                        hs.at[idxbuf.at[b, j]],
                        rowbuf.at[b, pl.ds(j * 128, 128)], gsems[b]).wait()

            def scatter_add(c, b):
                for j in range(4):
                    pltpu.async_copy(rowbuf.at[b, pl.ds(j * 128, 128)],
                                     acc_sh.at[idxbuf.at[b, 4 + j]], ssems[b],
                                     add=True)
                for j in range(4):
                    pltpu.make_async_copy(
                        rowbuf.at[b, pl.ds(j * 128, 128)],
                        acc_sh.at[idxbuf.at[b, 4 + j]], ssems[b]).wait()

            idx_wait(0, 0)
            gather(0, 0)

            @pl.loop(0, _CPT // 2)
            def _(k):
                for b in range(2):
                    c = 2 * k + b

                    @pl.when(c < _CPT - 1)
                    def _():
                        idx_wait(c + 1, 1 - b)
                        gather(c + 1, 1 - b)

                    gather_wait(c, b)
                    scatter_add(c, b)

                    @pl.when(c < _CPT - 2)
                    def _():
                        idx_load(c + 2, b)

        plsc.subcore_barrier()

        @pl.when(cid == 0)
        def _():
            run_edges(hs0)

        @pl.when(cid == 1)
        def _():
            run_edges(hs1)

        plsc.subcore_barrier()

        @pl.when(cid == 0)
        def _():
            _stage_out(acc_sh, acc0, t, r0)

        @pl.when(cid == 1)
        def _():
            _stage_out(acc_sh, acc1, t, r0)

    return scat


def _stage_out(acc_sh, acc, t, r0):
    @pl.when(t < 15)
    def _():
        pltpu.sync_copy(acc_sh.at[pl.ds(r0, _RPT)], acc.at[pl.ds(r0, _RPT)])

    @pl.when(t == 15)
    def _():
        pltpu.sync_copy(acc_sh.at[pl.ds(15 * _RPT, _LASTR)],
                        acc.at[pl.ds(15 * _RPT, _LASTR)])


def _make_degree():
    out16 = jax.ShapeDtypeStruct((_N, 16), jnp.float32)

    @functools.partial(
        pl.kernel,
        out_type=(out16, out16),
        mesh=_sc_mesh(),
        scratch_types=[
            pltpu.MemorySpace.VMEM_SHARED((_NPAD, 16), jnp.float32),
            pltpu.VMEM((_CPT // 2, 8, 128), jnp.int32),
            pltpu.VMEM((128, 16), jnp.float32),
            pltpu.SemaphoreType.DMA,
        ],
        compiler_params=_SC_PARAMS,
    )
    def degk(edges3, ones16, deg_a, deg_b, deg_sh, idxbuf, onesbuf, asem):
        t = lax.axis_index("s")
        cid = lax.axis_index("c")
        r0 = t * _RPT
        c0 = cid * (_CPT // 2)

        pltpu.sync_copy(ones16.at[pl.ds(r0, _RPT)], deg_sh.at[pl.ds(r0, _RPT)])
        pltpu.sync_copy(ones16.at[pl.ds(0, 128)], onesbuf)
        pltpu.sync_copy(edges3.at[t, pl.ds(c0, _CPT // 2)], idxbuf)
        plsc.subcore_barrier()

        @pl.loop(0, _CPT // 2)
        def _(k):
            for j in range(4):
                pltpu.async_copy(onesbuf, deg_sh.at[idxbuf.at[k, 4 + j]],
                                 asem, add=True)

        @pl.loop(0, _CPT // 2)
        def _(k):
            for j in range(4):
                pltpu.make_async_copy(
                    onesbuf, deg_sh.at[idxbuf.at[k, 4 + j]], asem).wait()

        plsc.subcore_barrier()

        @pl.when(cid == 0)
        def _():
            _stage_out(deg_sh, deg_a, t, r0)

        @pl.when(cid == 1)
        def _():
            _stage_out(deg_sh, deg_b, t, r0)

    return degk


def _row_spec(d):
    return pl.BlockSpec((_ROWB, d), lambda i: (i, 0))


def _full_spec(r, c):
    return pl.BlockSpec((r, c), lambda i: (0, 0))


def _t_first(x, w, deg_a, deg_b):
    def body(x_ref, w_ref, da_ref, db_ref, dinv_ref, hs0_ref, hs1_ref):
        deg = da_ref[...][:, 0:1] + db_ref[...][:, 0:1] - 1.0
        dcol = 1.0 / jnp.sqrt(deg)
        dinv_ref[...] = jnp.broadcast_to(dcol, (_ROWB, 16))
        h = jnp.dot(x_ref[...], w_ref[...], preferred_element_type=jnp.float32)
        hs = h * jnp.broadcast_to(dcol, (_ROWB, 128))
        hs0_ref[...] = hs[:, :64]
        hs1_ref[...] = hs[:, 64:]

    return pl.pallas_call(
        body,
        grid=(_GRID,),
        in_specs=[_row_spec(128), _full_spec(128, 128), _row_spec(16),
                  _row_spec(16)],
        out_specs=[_row_spec(16), _row_spec(64), _row_spec(64)],
        out_shape=[jax.ShapeDtypeStruct((_N, 16), jnp.float32),
                   jax.ShapeDtypeStruct((_N, 64), jnp.float32),
                   jax.ShapeDtypeStruct((_N, 64), jnp.float32)],
    )(x, w, deg_a, deg_b)


def _t_mid(a0, a1, dinv, bvec, res, w, d_next, emit_h):
    d_cur = 2 * a0.shape[1]
    hn = d_next // 2
    has_res = res is not None

    def body(*refs):
        if has_res:
            a0r, a1r, dvr, br, rr, wr = refs[:6]
            outs = refs[6:]
        else:
            a0r, a1r, dvr, br, wr = refs[:5]
            outs = refs[5:]
        u = jnp.concatenate([a0r[...], a1r[...]], axis=1)
        dv = jnp.broadcast_to(dvr[...][:, 0:1], (_ROWB, 128))
        h = jnp.maximum(dv[:, :d_cur] * u + br[...][0:1, :], 0.0)
        if has_res:
            h = h + rr[...]
        if emit_h:
            outs[0][...] = h
            o1, o2 = outs[1], outs[2]
        else:
            o1, o2 = outs[0], outs[1]
        hsn = jnp.dot(h, wr[...], preferred_element_type=jnp.float32)
        hsn = hsn * dv[:, :d_next]
        o1[...] = hsn[:, :hn]
        o2[...] = hsn[:, hn:]

    hd = d_cur // 2
    in_specs = [_row_spec(hd), _row_spec(hd),
                _row_spec(16), _full_spec(8, d_cur)]
    args = [a0, a1, dinv, bvec]
    if has_res:
        in_specs.append(_row_spec(d_cur))
        args.append(res)
    in_specs.append(_full_spec(d_cur, d_next))
    args.append(w)

    out_specs = []
    out_shape = []
    if emit_h:
        out_specs.append(_row_spec(d_cur))
        out_shape.append(jax.ShapeDtypeStruct((_N, d_cur), jnp.float32))
    out_specs += [_row_spec(hn), _row_spec(hn)]
    out_shape += [jax.ShapeDtypeStruct((_N, hn), jnp.float32)] * 2

    return pl.pallas_call(
        body,
        grid=(_GRID,),
        in_specs=in_specs,
        out_specs=out_specs,
        out_shape=out_shape,
    )(*args)


def _t_mha(a0, a1, dinv, be4, wv, bv, wo, bo, wd1):
    def body(a0r, a1r, dvr, ber, wvr, bvr, wor, bor, wdr, z_ref, o1, o2):
        u = jnp.concatenate([a0r[...], a1r[...]], axis=1)
        dv = jnp.broadcast_to(dvr[...][:, 0:1], (_ROWB, 128))
        z = dv[:, :64] * u + ber[...][0:1, :]
        z_ref[...] = z
        za = jnp.dot(z, wvr[...], preferred_element_type=jnp.float32)
        za = za + bvr[...][0:1, :]
        za = jnp.dot(za, wor[...], preferred_element_type=jnp.float32)
        za = za + bor[...][0:1, :]
        hs = jnp.dot(za, wdr[...], preferred_element_type=jnp.float32) * dv
        o1[...] = hs[:, :64]
        o2[...] = hs[:, 64:]

    return pl.pallas_call(
        body,
        grid=(_GRID,),
        in_specs=[_row_spec(32), _row_spec(32),
                  _row_spec(16), _full_spec(8, 64), _full_spec(64, 64),
                  _full_spec(8, 64), _full_spec(64, 64), _full_spec(8, 64),
                  _full_spec(64, 128)],
        out_specs=[_row_spec(64), _row_spec(64), _row_spec(64)],
        out_shape=[jax.ShapeDtypeStruct((_N, 64), jnp.float32),
                   jax.ShapeDtypeStruct((_N, 64), jnp.float32),
                   jax.ShapeDtypeStruct((_N, 64), jnp.float32)],
    )(a0, a1, dinv, be4, wv, bv, wo, bo, wd1)


def _t_last(a0, a1, dinv, bvec):
    def body(a0r, a1r, dvr, br, out_ref):
        u = jnp.concatenate([a0r[...], a1r[...]], axis=1)
        dv = jnp.broadcast_to(dvr[...][:, 0:1], (_ROWB, 128))
        out_ref[...] = dv * u + br[...][0:1, :]

    return pl.pallas_call(
        body,
        grid=(_GRID,),
        in_specs=[_row_spec(64), _row_spec(64),
                  _row_spec(16), _full_spec(8, 128)],
        out_specs=_row_spec(128),
        out_shape=jax.ShapeDtypeStruct((_N, 128), jnp.float32),
    )(a0, a1, dinv, bvec)


def _bb(b):
    return jnp.broadcast_to(b.reshape(1, -1), (8, b.shape[0]))


def kernel(x, edge_index, W_e1, b_e1, W_e2, b_e2, W_e3, b_e3, W_e4, b_e4,
           W_d1, b_d1, W_d2, b_d2, W_d3, b_d3, W_d4, b_d4,
           W_qkv, b_qkv, W_o, b_o):
    src = edge_index[0]
    dst = edge_index[1]
    npad_e = _EPAD // _TILES - _E // _TILES
    ar = jnp.arange(_TILES * npad_e, dtype=jnp.int32)
    pad_s = ((ar * 131) % _N).reshape(_TILES, npad_e)
    pad_d = (_N + (ar * 7) % (_NPAD - _N)).reshape(_TILES, npad_e)
    s2 = jnp.concatenate([src.reshape(_TILES, -1), pad_s], axis=1)
    d2 = jnp.concatenate([dst.reshape(_TILES, -1), pad_d], axis=1)
    edges3 = jnp.concatenate(
        [s2.reshape(_TILES, _CPT, 4, 128), d2.reshape(_TILES, _CPT, 4, 128)],
        axis=2)

    ones16 = jnp.ones((_NPAD, 16), jnp.float32)

    deg_a, deg_b = _make_degree()(edges3, ones16)
    dinv, hs0, hs1 = _t_first(x, W_e1, deg_a, deg_b)

    scat64 = _make_scatter(64)
    scat32 = _make_scatter(32)

    a0, a1 = scat64(hs0, hs1, edges3)
    h1, hs0, hs1 = _t_mid(a0, a1, dinv, _bb(b_e1), None, W_e2, 128, True)
    a0, a1 = scat64(hs0, hs1, edges3)
    h2, hs0, hs1 = _t_mid(a0, a1, dinv, _bb(b_e2), h1, W_e3, 128, True)
    a0, a1 = scat64(hs0, hs1, edges3)
    hs0, hs1 = _t_mid(a0, a1, dinv, _bb(b_e3), h2, W_e4, 64, False)
    a0, a1 = scat32(hs0, hs1, edges3)
    z, hs0, hs1 = _t_mha(a0, a1, dinv, _bb(b_e4),
                         W_qkv[:, 128:], _bb(b_qkv[128:]), W_o, _bb(b_o),
                         W_d1)
    a0, a1 = scat64(hs0, hs1, edges3)
    g1, hs0, hs1 = _t_mid(a0, a1, dinv, _bb(b_d1), None, W_d2, 128, True)
    a0, a1 = scat64(hs0, hs1, edges3)
    g2, hs0, hs1 = _t_mid(a0, a1, dinv, _bb(b_d2), g1, W_d3, 128, True)
    a0, a1 = scat64(hs0, hs1, edges3)
    hs0, hs1 = _t_mid(a0, a1, dinv, _bb(b_d3), g2, W_d4, 128, False)
    a0, a1 = scat64(hs0, hs1, edges3)
    x_recon = _t_last(a0, a1, dinv, _bb(b_d4))
    return (x_recon, z)
